# Initial kernel scaffold; baseline (speedup 1.0000x reference)
import jax
import jax.numpy as jnp
from jax.experimental import pallas as pl


def kernel(node_type, edge_index, edge_type, batch, node_emb, W_rel, W_root, conv_bias, risk_W1, risk_b1, risk_W2, risk_b2, safe_W1, safe_b1, safe_W2, safe_b2):
    raise NotImplementedError("write your pallas kernel here")


# speedup vs baseline: 8.7952x; 8.7952x over previous
import functools

import jax
import jax.numpy as jnp
from jax import lax
from jax.experimental import pallas as pl
from jax.experimental.pallas import tpu as pltpu
from jax.experimental.pallas import tpu_sc as plsc

N = 10000
E = 320000
D = 128
R = 8
L = 3
B = 16

NC = 2
NS = 16
LANES = 16

EDGE_CHUNK = 80
E_PER_TILE = E // (NC * NS)
N_CHUNKS_AGG = E_PER_TILE // EDGE_CHUNK
E_PER_CNT_TILE = E // NS
N_CHUNKS_CNT = E_PER_CNT_TILE // EDGE_CHUNK
ROW_CHUNK = 80
N_ROW_CHUNKS = N // ROW_CHUNK
GATHER_CHUNKS = N // EDGE_CHUNK

_MESH = plsc.VectorSubcoreMesh(
    core_axis_name="c", subcore_axis_name="s", num_cores=NC, num_subcores=NS)


@functools.partial(
    pl.kernel,
    out_type=(
        jax.ShapeDtypeStruct((N, D), jnp.float32),
        jax.ShapeDtypeStruct((N * R, D), jnp.float32),
    ),
    mesh=_MESH,
    scratch_types=(
        pltpu.VMEM((EDGE_CHUNK,), jnp.int32),
        pltpu.VMEM((EDGE_CHUNK, D), jnp.float32),
        pltpu.VMEM((EDGE_CHUNK,), jnp.int32),
        pltpu.VMEM((EDGE_CHUNK,), jnp.int32),
        pltpu.VMEM((EDGE_CHUNK, D), jnp.float32),
        pltpu.VMEM((EDGE_CHUNK, D), jnp.float32),
        pltpu.VMEM((EDGE_CHUNK, D), jnp.float32),
        pltpu.VMEM_SHARED((N, D), jnp.float32),
        pltpu.SemaphoreType.DMA,
    ),
)
def _sc_embed_and_count(node_type_hbm, edge_type_hbm, dst_hbm, oh8_hbm,
                        emb_hbm, x0_hbm, invc_hbm,
                        idx_v, rows_v, ti_v, di_v, oh_v, c_v, t2_v,
                        acc_sh, sem):
  c = lax.axis_index("c")
  s = lax.axis_index("s")

  @pl.when(c == 1)
  def _embed():
    def chunk_body(k, _):
      cid = s + k * NS

      @pl.when(cid < GATHER_CHUNKS)
      def _():
        base = pl.multiple_of(cid * EDGE_CHUNK, 8)
        pltpu.sync_copy(node_type_hbm.at[pl.ds(base, EDGE_CHUNK)], idx_v)
        pltpu.async_copy(emb_hbm.at[idx_v], rows_v, sem).wait()
        pltpu.sync_copy(rows_v, x0_hbm.at[pl.ds(base, EDGE_CHUNK)])
      return 0

    lax.fori_loop(0, (GATHER_CHUNKS + NS - 1) // NS, chunk_body, 0)

  @pl.when(c == 0)
  def _zero():
    for j in range(ROW_CHUNK):
      for cc in range(D // LANES):
        t2_v[j, pl.ds(cc * LANES, LANES)] = jnp.zeros((LANES,), jnp.float32)

    def zero_body(k, _):
      cid = s + k * NS

      @pl.when(cid < N_ROW_CHUNKS)
      def _():
        pltpu.sync_copy(t2_v, acc_sh.at[pl.ds(cid * ROW_CHUNK, ROW_CHUNK)])
      return 0

    lax.fori_loop(0, (N_ROW_CHUNKS + NS - 1) // NS, zero_body, 0)

  plsc.subcore_barrier()

  @pl.when(c == 0)
  def _count():
    def chunk_body(k, _):
      base = pl.multiple_of(s * E_PER_CNT_TILE + k * EDGE_CHUNK, 8)
      pltpu.sync_copy(edge_type_hbm.at[pl.ds(base, EDGE_CHUNK)], ti_v)
      pltpu.sync_copy(dst_hbm.at[pl.ds(base, EDGE_CHUNK)], di_v)
      pltpu.async_copy(oh8_hbm.at[ti_v], oh_v, sem).wait()
      pltpu.sync_copy(oh_v, acc_sh.at[di_v], add=True)
      return 0

    lax.fori_loop(0, N_CHUNKS_CNT, chunk_body, 0)

  plsc.subcore_barrier()

  @pl.when(c == 0)
  def _recip():
    def wb_body(k, _):
      cid = s + k * NS

      @pl.when(cid < N_ROW_CHUNKS)
      def _():
        pltpu.sync_copy(acc_sh.at[pl.ds(cid * ROW_CHUNK, ROW_CHUNK)], c_v)

        def batch_body(b, _):
          for j2 in range(ROW_CHUNK):
            src_row = b * (ROW_CHUNK // R) + j2 // R
            blk = pl.ds((j2 % R) * LANES, LANES)
            rec = 1.0 / jnp.maximum(c_v[src_row, blk], 1.0)
            for cc in range(D // LANES):
              t2_v[j2, pl.ds(cc * LANES, LANES)] = rec
          obase = pl.multiple_of(cid * ROW_CHUNK * R + b * ROW_CHUNK, 8)
          pltpu.sync_copy(t2_v, invc_hbm.at[pl.ds(obase, ROW_CHUNK)])
          return 0

        lax.fori_loop(0, R, batch_body, 0)
      return 0

    lax.fori_loop(0, (N_ROW_CHUNKS + NS - 1) // NS, wb_body, 0)


@functools.partial(
    pl.kernel,
    out_type=jax.ShapeDtypeStruct((NC, N, D), jnp.float32),
    mesh=_MESH,
    scratch_types=(
        pltpu.VMEM((EDGE_CHUNK,), jnp.int32),
        pltpu.VMEM((EDGE_CHUNK,), jnp.int32),
        pltpu.VMEM((EDGE_CHUNK,), jnp.int32),
        pltpu.VMEM((EDGE_CHUNK, D), jnp.float32),
        pltpu.VMEM((EDGE_CHUNK, D), jnp.float32),
        pltpu.VMEM_SHARED((N, D), jnp.float32),
        pltpu.SemaphoreType.DMA,
        pltpu.SemaphoreType.DMA,
    ),
)
def _sc_aggregate(hflat_hbm, gidx_hbm, dst_hbm, widx_hbm, invc_hbm, zrows_hbm,
                  out_hbm,
                  gi_v, di_v, wi_v, wr_v, rows_v, acc_sh, sem, sem2):
  c = lax.axis_index("c")
  s = lax.axis_index("s")
  wid = s * NC + c

  pltpu.sync_copy(zrows_hbm, rows_v)

  def zero_body(k, _):
    cid = s + k * NS

    @pl.when(cid < N_ROW_CHUNKS)
    def _():
      pltpu.sync_copy(rows_v, acc_sh.at[pl.ds(cid * ROW_CHUNK, ROW_CHUNK)])
    return 0

  lax.fori_loop(0, (N_ROW_CHUNKS + NS - 1) // NS, zero_body, 0)
  plsc.subcore_barrier()

  def chunk_body(k, _):
    base = pl.multiple_of(wid * E_PER_TILE + k * EDGE_CHUNK, 8)
    pltpu.sync_copy(gidx_hbm.at[pl.ds(base, EDGE_CHUNK)], gi_v)
    pltpu.sync_copy(dst_hbm.at[pl.ds(base, EDGE_CHUNK)], di_v)
    pltpu.sync_copy(widx_hbm.at[pl.ds(base, EDGE_CHUNK)], wi_v)
    wcopy = pltpu.async_copy(invc_hbm.at[wi_v], wr_v, sem2)
    pltpu.async_copy(hflat_hbm.at[gi_v], rows_v, sem).wait()
    wcopy.wait()

    for j in range(EDGE_CHUNK):
      for cc in range(D // LANES):
        sl = pl.ds(cc * LANES, LANES)
        rows_v[j, sl] = rows_v[j, sl] * wr_v[j, sl]
    pltpu.sync_copy(rows_v, acc_sh.at[di_v], add=True)
    return 0

  lax.fori_loop(0, N_CHUNKS_AGG, chunk_body, 0)
  plsc.subcore_barrier()

  def wb_body(k, _):
    cid = s + k * NS

    @pl.when(cid < N_ROW_CHUNKS)
    def _():
      sl = pl.ds(cid * ROW_CHUNK, ROW_CHUNK)
      pltpu.sync_copy(acc_sh.at[sl], rows_v)

      @pl.when(c == 0)
      def _():
        pltpu.sync_copy(rows_v, out_hbm.at[0, sl])

      @pl.when(c == 1)
      def _():
        pltpu.sync_copy(rows_v, out_hbm.at[1, sl])
    return 0

  lax.fori_loop(0, (N_ROW_CHUNKS + NS - 1) // NS, wb_body, 0)


ROW_BLK = 1000


def _mm_body(x_ref, w_ref, b_ref, out_ref):
  i = pl.program_id(0)
  h = jnp.dot(x_ref[...], w_ref[0], preferred_element_type=jnp.float32)
  out_ref[0] = h + jnp.where(i == 0, b_ref[...], 0.0)


def _tc_project(x, w_all, bias_row):
  return pl.pallas_call(
      _mm_body,
      grid=(R + 1, N // ROW_BLK),
      in_specs=[
          pl.BlockSpec((ROW_BLK, D), lambda i, j: (j, 0)),
          pl.BlockSpec((1, D, D), lambda i, j: (i, 0, 0)),
          pl.BlockSpec((1, D), lambda i, j: (0, 0)),
      ],
      out_specs=pl.BlockSpec((1, ROW_BLK, D), lambda i, j: (i, j, 0)),
      out_shape=jax.ShapeDtypeStruct((R + 1, N, D), jnp.float32),
  )(x, w_all, bias_row)


def _combine_body(h_ref, a_ref, out_ref):
  out_ref[...] = jnp.maximum(h_ref[0] + a_ref[0] + a_ref[1], 0.0)


def _tc_combine(h9, partials):
  return pl.pallas_call(
      _combine_body,
      grid=(N // ROW_BLK,),
      in_specs=[
          pl.BlockSpec((1, ROW_BLK, D), lambda j: (0, j, 0)),
          pl.BlockSpec((NC, ROW_BLK, D), lambda j: (0, j, 0)),
      ],
      out_specs=pl.BlockSpec((ROW_BLK, D), lambda j: (j, 0)),
      out_shape=jax.ShapeDtypeStruct((N, D), jnp.float32),
  )(h9, partials)


def _pool_heads_body(batch_ref, x_ref, rw1_ref, rb1_ref, rw2_ref, rb2_ref,
                     sw1_ref, sb1_ref, sw2_ref, sb2_ref,
                     risk_ref, safe_ref, g_acc, c_acc):
  j = pl.program_id(0)

  @pl.when(j == 0)
  def _():
    g_acc[...] = jnp.zeros_like(g_acc)
    c_acc[...] = jnp.zeros_like(c_acc)

  bids = batch_ref[0, 0, :]
  sel = (lax.broadcasted_iota(jnp.int32, (B, ROW_BLK), 0)
         == bids[None, :]).astype(jnp.float32)
  g_acc[...] += jnp.dot(sel, x_ref[...], preferred_element_type=jnp.float32)
  c_acc[...] += jnp.broadcast_to(
      jnp.sum(sel, axis=1, keepdims=True), (B, D))

  @pl.when(j == pl.num_programs(0) - 1)
  def _():
    g = g_acc[...] / jnp.maximum(c_acc[...], 1.0)
    r1 = jnp.maximum(
        jnp.dot(g, rw1_ref[...], preferred_element_type=jnp.float32)
        + rb1_ref[...], 0.0)
    risk_ref[...] = jnp.dot(
        r1, rw2_ref[...], preferred_element_type=jnp.float32) + rb2_ref[...]
    s1 = jnp.maximum(
        jnp.dot(g, sw1_ref[...], preferred_element_type=jnp.float32)
        + sb1_ref[...], 0.0)
    safe_ref[...] = jnp.dot(
        s1, sw2_ref[...], preferred_element_type=jnp.float32) + sb2_ref[...]


def _tc_pool_heads(batch3d, x, rw1, rb1, rw2p, rb2f, sw1, sb1, sw2p, sb2f):
  full = lambda shape: pl.BlockSpec(shape, lambda j: tuple(0 for _ in shape))
  return pl.pallas_call(
      _pool_heads_body,
      grid=(N // ROW_BLK,),
      in_specs=[
          pl.BlockSpec((1, 1, ROW_BLK), lambda j: (j, 0, 0)),
          pl.BlockSpec((ROW_BLK, D), lambda j: (j, 0)),
          full((D, D)), full((1, D)), full((D, D)), full((1, D)),
          full((D, D)), full((1, D)), full((D, D)), full((1, D)),
      ],
      out_specs=[full((B, D)), full((B, D))],
      out_shape=[
          jax.ShapeDtypeStruct((B, D), jnp.float32),
          jax.ShapeDtypeStruct((B, D), jnp.float32),
      ],
      scratch_shapes=[
          pltpu.VMEM((B, D), jnp.float32),
          pltpu.VMEM((B, D), jnp.float32),
      ],
  )(batch3d, x, rw1, rb1, rw2p, rb2f, sw1, sb1, sw2p, sb2f)


def kernel(node_type, edge_index, edge_type, batch, node_emb, W_rel, W_root,
           conv_bias, risk_W1, risk_b1, risk_W2, risk_b2, safe_W1, safe_b1,
           safe_W2, safe_b2):
  node_type = node_type.astype(jnp.int32)
  edge_type = edge_type.astype(jnp.int32)
  src = edge_index[0].astype(jnp.int32)
  dst = edge_index[1].astype(jnp.int32)
  gidx = (edge_type + 1) * N + src
  widx = dst * R + edge_type

  zrows_d = jnp.zeros((ROW_CHUNK, D), jnp.float32)
  oh8 = (jnp.arange(D, dtype=jnp.int32)[None, :] // LANES
         == jnp.arange(R, dtype=jnp.int32)[:, None]).astype(jnp.float32)

  x0, invc = _sc_embed_and_count(
      node_type, edge_type, dst, oh8, node_emb.astype(jnp.float32))

  x = x0
  for l in range(L):
    w_all = jnp.concatenate([W_root[l][None], W_rel[l]], axis=0)
    h9 = _tc_project(x, w_all, conv_bias[l][None, :])
    partials = _sc_aggregate(h9.reshape((R + 1) * N, D), gidx, dst, widx,
                             invc, zrows_d)
    x = _tc_combine(h9, partials)

  batch3d = batch.astype(jnp.int32).reshape(N // ROW_BLK, 1, ROW_BLK)
  pad = jnp.zeros((D, D - 1), jnp.float32)
  rw2p = jnp.concatenate([risk_W2, pad], axis=1)
  sw2p = jnp.concatenate([safe_W2, pad], axis=1)
  rb2f = jnp.full((1, D), risk_b2[0], jnp.float32)
  sb2f = jnp.full((1, D), safe_b2[0], jnp.float32)
  risk, safe = _tc_pool_heads(batch3d, x, risk_W1, risk_b1[None, :], rw2p,
                              rb2f, safe_W1, safe_b1[None, :], sw2p, sb2f)
  return risk[:, 0], safe[:, 0]


# speedup vs baseline: 12.1348x; 1.3797x over previous
import functools

import jax
import jax.numpy as jnp
from jax import lax
from jax.experimental import pallas as pl
from jax.experimental.pallas import tpu as pltpu
from jax.experimental.pallas import tpu_sc as plsc

N = 10000
E = 320000
D = 128
R = 8
L = 3
B = 16

NC = 2
NS = 16
LANES = 16

EDGE_CHUNK = 80
E_PER_TILE = E // (NC * NS)
N_CHUNKS_AGG = E_PER_TILE // EDGE_CHUNK
E_PER_CNT_TILE = E // NS
N_CHUNKS_CNT = E_PER_CNT_TILE // EDGE_CHUNK
ROW_CHUNK = 80
N_ROW_CHUNKS = N // ROW_CHUNK
GATHER_CHUNKS = N // EDGE_CHUNK

_MESH = plsc.VectorSubcoreMesh(
    core_axis_name="c", subcore_axis_name="s", num_cores=NC, num_subcores=NS)


@functools.partial(
    pl.kernel,
    out_type=(
        jax.ShapeDtypeStruct((N, D), jnp.float32),
        jax.ShapeDtypeStruct((N * R, D), jnp.float32),
    ),
    mesh=_MESH,
    scratch_types=(
        pltpu.VMEM((EDGE_CHUNK,), jnp.int32),
        pltpu.VMEM((EDGE_CHUNK, D), jnp.float32),
        pltpu.VMEM((EDGE_CHUNK,), jnp.int32),
        pltpu.VMEM((EDGE_CHUNK,), jnp.int32),
        pltpu.VMEM((EDGE_CHUNK, D), jnp.float32),
        pltpu.VMEM((EDGE_CHUNK, D), jnp.float32),
        pltpu.VMEM((EDGE_CHUNK, D), jnp.float32),
        pltpu.VMEM_SHARED((N, D), jnp.float32),
        pltpu.SemaphoreType.DMA,
    ),
)
def _sc_embed_and_count(node_type_hbm, edge_type_hbm, dst_hbm, oh8_hbm,
                        emb_hbm, x0_hbm, invc_hbm,
                        idx_v, rows_v, ti_v, di_v, oh_v, c_v, t2_v,
                        acc_sh, sem):
  c = lax.axis_index("c")
  s = lax.axis_index("s")

  @pl.when(c == 1)
  def _embed():
    def chunk_body(k, _):
      cid = s + k * NS

      @pl.when(cid < GATHER_CHUNKS)
      def _():
        base = pl.multiple_of(cid * EDGE_CHUNK, 8)
        pltpu.sync_copy(node_type_hbm.at[pl.ds(base, EDGE_CHUNK)], idx_v)
        pltpu.async_copy(emb_hbm.at[idx_v], rows_v, sem).wait()
        pltpu.sync_copy(rows_v, x0_hbm.at[pl.ds(base, EDGE_CHUNK)])
      return 0

    lax.fori_loop(0, (GATHER_CHUNKS + NS - 1) // NS, chunk_body, 0)

  @pl.when(c == 0)
  def _zero():
    for j in range(ROW_CHUNK):
      for cc in range(D // LANES):
        t2_v[j, pl.ds(cc * LANES, LANES)] = jnp.zeros((LANES,), jnp.float32)

    def zero_body(k, _):
      cid = s + k * NS

      @pl.when(cid < N_ROW_CHUNKS)
      def _():
        pltpu.sync_copy(t2_v, acc_sh.at[pl.ds(cid * ROW_CHUNK, ROW_CHUNK)])
      return 0

    lax.fori_loop(0, (N_ROW_CHUNKS + NS - 1) // NS, zero_body, 0)

  plsc.subcore_barrier()

  @pl.when(c == 0)
  def _count():
    def chunk_body(k, _):
      base = pl.multiple_of(s * E_PER_CNT_TILE + k * EDGE_CHUNK, 8)
      pltpu.sync_copy(edge_type_hbm.at[pl.ds(base, EDGE_CHUNK)], ti_v)
      pltpu.sync_copy(dst_hbm.at[pl.ds(base, EDGE_CHUNK)], di_v)
      pltpu.async_copy(oh8_hbm.at[ti_v], oh_v, sem).wait()
      pltpu.sync_copy(oh_v, acc_sh.at[di_v], add=True)
      return 0

    lax.fori_loop(0, N_CHUNKS_CNT, chunk_body, 0)

  plsc.subcore_barrier()

  @pl.when(c == 0)
  def _recip():
    def wb_body(k, _):
      cid = s + k * NS

      @pl.when(cid < N_ROW_CHUNKS)
      def _():
        pltpu.sync_copy(acc_sh.at[pl.ds(cid * ROW_CHUNK, ROW_CHUNK)], c_v)

        def batch_body(b, _):
          for j2 in range(ROW_CHUNK):
            src_row = b * (ROW_CHUNK // R) + j2 // R
            blk = pl.ds((j2 % R) * LANES, LANES)
            rec = 1.0 / jnp.maximum(c_v[src_row, blk], 1.0)
            for cc in range(D // LANES):
              t2_v[j2, pl.ds(cc * LANES, LANES)] = rec
          obase = pl.multiple_of(cid * ROW_CHUNK * R + b * ROW_CHUNK, 8)
          pltpu.sync_copy(t2_v, invc_hbm.at[pl.ds(obase, ROW_CHUNK)])
          return 0

        lax.fori_loop(0, R, batch_body, 0)
      return 0

    lax.fori_loop(0, (N_ROW_CHUNKS + NS - 1) // NS, wb_body, 0)


@functools.partial(
    pl.kernel,
    out_type=jax.ShapeDtypeStruct((NC, N, D), jnp.float32),
    mesh=_MESH,
    scratch_types=(
        pltpu.VMEM((EDGE_CHUNK,), jnp.int32),
        pltpu.VMEM((EDGE_CHUNK,), jnp.int32),
        pltpu.VMEM((EDGE_CHUNK,), jnp.int32),
        pltpu.VMEM((EDGE_CHUNK,), jnp.int32),
        pltpu.VMEM((EDGE_CHUNK,), jnp.int32),
        pltpu.VMEM((EDGE_CHUNK,), jnp.int32),
        pltpu.VMEM((EDGE_CHUNK, D), jnp.float32),
        pltpu.VMEM((EDGE_CHUNK, D), jnp.float32),
        pltpu.VMEM((EDGE_CHUNK, D), jnp.float32),
        pltpu.VMEM((EDGE_CHUNK, D), jnp.float32),
        pltpu.VMEM_SHARED((N, D), jnp.float32),
        pltpu.SemaphoreType.DMA,
        pltpu.SemaphoreType.DMA,
        pltpu.SemaphoreType.DMA,
        pltpu.SemaphoreType.DMA,
    ),
)
def _sc_aggregate(hflat_hbm, gidx_hbm, dst_hbm, widx_hbm, invc_hbm, zrows_hbm,
                  out_hbm,
                  gi_v0, gi_v1, di_v0, di_v1, wi_v0, wi_v1, wr_v0, wr_v1,
                  rows_v0, rows_v1, acc_sh, semi0, semi1, semg0, semg1):
  c = lax.axis_index("c")
  s = lax.axis_index("s")
  wid = s * NC + c

  pltpu.sync_copy(zrows_hbm, rows_v0)

  def zero_body(k, _):
    cid = s + k * NS

    @pl.when(cid < N_ROW_CHUNKS)
    def _():
      pltpu.sync_copy(rows_v0, acc_sh.at[pl.ds(cid * ROW_CHUNK, ROW_CHUNK)])
    return 0

  lax.fori_loop(0, (N_ROW_CHUNKS + NS - 1) // NS, zero_body, 0)
  plsc.subcore_barrier()

  bufs = ((gi_v0, di_v0, wi_v0, wr_v0, rows_v0, semi0, semg0),
          (gi_v1, di_v1, wi_v1, wr_v1, rows_v1, semi1, semg1))

  def fire_idx(k, b):
    gi, di, wi, _, _, semi, _ = bufs[b]
    base = pl.multiple_of(wid * E_PER_TILE + k * EDGE_CHUNK, 8)
    return (pltpu.async_copy(gidx_hbm.at[pl.ds(base, EDGE_CHUNK)], gi, semi),
            pltpu.async_copy(dst_hbm.at[pl.ds(base, EDGE_CHUNK)], di, semi),
            pltpu.async_copy(widx_hbm.at[pl.ds(base, EDGE_CHUNK)], wi, semi))

  def fire_gather(b):
    gi, _, wi, wr, rows, _, semg = bufs[b]
    return (pltpu.async_copy(hflat_hbm.at[gi], rows, semg),
            pltpu.async_copy(invc_hbm.at[wi], wr, semg))

  def finish(b):
    _, di, _, wr, rows, _, _ = bufs[b]

    def scale_body(jj, _):
      for j2 in range(8):
        j = jj * 8 + j2
        for cc in range(D // LANES):
          sl = pl.ds(cc * LANES, LANES)
          rows[j, sl] = rows[j, sl] * wr[j, sl]
      return 0

    lax.fori_loop(0, EDGE_CHUNK // 8, scale_body, 0)
    pltpu.sync_copy(rows, acc_sh.at[di], add=True)

  def pair_body(m, _):
    k0 = m * 2
    i0 = fire_idx(k0, 0)
    i1 = fire_idx(k0 + 1, 1)
    for dsc in i0:
      dsc.wait()
    g0 = fire_gather(0)
    for dsc in i1:
      dsc.wait()
    g1 = fire_gather(1)
    for dsc in g0:
      dsc.wait()
    finish(0)
    for dsc in g1:
      dsc.wait()
    finish(1)
    return 0

  lax.fori_loop(0, N_CHUNKS_AGG // 2, pair_body, 0)
  for dsc in fire_idx(N_CHUNKS_AGG - 1, 0):
    dsc.wait()
  for dsc in fire_gather(0):
    dsc.wait()
  finish(0)

  plsc.subcore_barrier()

  def wb_body(k, _):
    cid = s + k * NS

    @pl.when(cid < N_ROW_CHUNKS)
    def _():
      sl = pl.ds(cid * ROW_CHUNK, ROW_CHUNK)
      pltpu.sync_copy(acc_sh.at[sl], rows_v0)

      @pl.when(c == 0)
      def _():
        pltpu.sync_copy(rows_v0, out_hbm.at[0, sl])

      @pl.when(c == 1)
      def _():
        pltpu.sync_copy(rows_v0, out_hbm.at[1, sl])
    return 0

  lax.fori_loop(0, (N_ROW_CHUNKS + NS - 1) // NS, wb_body, 0)


ROW_BLK = 1000


def _mm_body(x_ref, w_ref, b_ref, out_ref):
  i = pl.program_id(0)
  h = jnp.dot(x_ref[...], w_ref[0], preferred_element_type=jnp.float32)
  out_ref[0] = h + jnp.where(i == 0, b_ref[...], 0.0)


def _tc_project(x, w_all, bias_row):
  return pl.pallas_call(
      _mm_body,
      grid=(R + 1, N // ROW_BLK),
      in_specs=[
          pl.BlockSpec((ROW_BLK, D), lambda i, j: (j, 0)),
          pl.BlockSpec((1, D, D), lambda i, j: (i, 0, 0)),
          pl.BlockSpec((1, D), lambda i, j: (0, 0)),
      ],
      out_specs=pl.BlockSpec((1, ROW_BLK, D), lambda i, j: (i, j, 0)),
      out_shape=jax.ShapeDtypeStruct((R + 1, N, D), jnp.float32),
  )(x, w_all, bias_row)


def _combine_body(h_ref, a_ref, out_ref):
  out_ref[...] = jnp.maximum(h_ref[0] + a_ref[0] + a_ref[1], 0.0)


def _tc_combine(h9, partials):
  return pl.pallas_call(
      _combine_body,
      grid=(N // ROW_BLK,),
      in_specs=[
          pl.BlockSpec((1, ROW_BLK, D), lambda j: (0, j, 0)),
          pl.BlockSpec((NC, ROW_BLK, D), lambda j: (0, j, 0)),
      ],
      out_specs=pl.BlockSpec((ROW_BLK, D), lambda j: (j, 0)),
      out_shape=jax.ShapeDtypeStruct((N, D), jnp.float32),
  )(h9, partials)


def _pool_heads_body(batch_ref, x_ref, rw1_ref, rb1_ref, rw2_ref, rb2_ref,
                     sw1_ref, sb1_ref, sw2_ref, sb2_ref,
                     risk_ref, safe_ref, g_acc, c_acc):
  j = pl.program_id(0)

  @pl.when(j == 0)
  def _():
    g_acc[...] = jnp.zeros_like(g_acc)
    c_acc[...] = jnp.zeros_like(c_acc)

  bids = batch_ref[0, 0, :]
  sel = (lax.broadcasted_iota(jnp.int32, (B, ROW_BLK), 0)
         == bids[None, :]).astype(jnp.float32)
  g_acc[...] += jnp.dot(sel, x_ref[...], preferred_element_type=jnp.float32)
  c_acc[...] += jnp.broadcast_to(
      jnp.sum(sel, axis=1, keepdims=True), (B, D))

  @pl.when(j == pl.num_programs(0) - 1)
  def _():
    g = g_acc[...] / jnp.maximum(c_acc[...], 1.0)
    r1 = jnp.maximum(
        jnp.dot(g, rw1_ref[...], preferred_element_type=jnp.float32)
        + rb1_ref[...], 0.0)
    risk_ref[...] = jnp.dot(
        r1, rw2_ref[...], preferred_element_type=jnp.float32) + rb2_ref[...]
    s1 = jnp.maximum(
        jnp.dot(g, sw1_ref[...], preferred_element_type=jnp.float32)
        + sb1_ref[...], 0.0)
    safe_ref[...] = jnp.dot(
        s1, sw2_ref[...], preferred_element_type=jnp.float32) + sb2_ref[...]


def _tc_pool_heads(batch3d, x, rw1, rb1, rw2p, rb2f, sw1, sb1, sw2p, sb2f):
  full = lambda shape: pl.BlockSpec(shape, lambda j: tuple(0 for _ in shape))
  return pl.pallas_call(
      _pool_heads_body,
      grid=(N // ROW_BLK,),
      in_specs=[
          pl.BlockSpec((1, 1, ROW_BLK), lambda j: (j, 0, 0)),
          pl.BlockSpec((ROW_BLK, D), lambda j: (j, 0)),
          full((D, D)), full((1, D)), full((D, D)), full((1, D)),
          full((D, D)), full((1, D)), full((D, D)), full((1, D)),
      ],
      out_specs=[full((B, D)), full((B, D))],
      out_shape=[
          jax.ShapeDtypeStruct((B, D), jnp.float32),
          jax.ShapeDtypeStruct((B, D), jnp.float32),
      ],
      scratch_shapes=[
          pltpu.VMEM((B, D), jnp.float32),
          pltpu.VMEM((B, D), jnp.float32),
      ],
  )(batch3d, x, rw1, rb1, rw2p, rb2f, sw1, sb1, sw2p, sb2f)


def kernel(node_type, edge_index, edge_type, batch, node_emb, W_rel, W_root,
           conv_bias, risk_W1, risk_b1, risk_W2, risk_b2, safe_W1, safe_b1,
           safe_W2, safe_b2):
  node_type = node_type.astype(jnp.int32)
  edge_type = edge_type.astype(jnp.int32)
  src = edge_index[0].astype(jnp.int32)
  dst = edge_index[1].astype(jnp.int32)
  gidx = (edge_type + 1) * N + src
  widx = dst * R + edge_type

  zrows_d = jnp.zeros((ROW_CHUNK, D), jnp.float32)
  oh8 = (jnp.arange(D, dtype=jnp.int32)[None, :] // LANES
         == jnp.arange(R, dtype=jnp.int32)[:, None]).astype(jnp.float32)

  x0, invc = _sc_embed_and_count(
      node_type, edge_type, dst, oh8, node_emb.astype(jnp.float32))

  x = x0
  for l in range(L):
    w_all = jnp.concatenate([W_root[l][None], W_rel[l]], axis=0)
    h9 = _tc_project(x, w_all, conv_bias[l][None, :])
    partials = _sc_aggregate(h9.reshape((R + 1) * N, D), gidx, dst, widx,
                             invc, zrows_d)
    x = _tc_combine(h9, partials)

  batch3d = batch.astype(jnp.int32).reshape(N // ROW_BLK, 1, ROW_BLK)
  pad = jnp.zeros((D, D - 1), jnp.float32)
  rw2p = jnp.concatenate([risk_W2, pad], axis=1)
  sw2p = jnp.concatenate([safe_W2, pad], axis=1)
  rb2f = jnp.full((1, D), risk_b2[0], jnp.float32)
  sb2f = jnp.full((1, D), safe_b2[0], jnp.float32)
  risk, safe = _tc_pool_heads(batch3d, x, risk_W1, risk_b1[None, :], rw2p,
                              rb2f, safe_W1, safe_b1[None, :], sw2p, sb2f)
  return risk[:, 0], safe[:, 0]


# speedup vs baseline: 12.4512x; 1.0261x over previous
import functools

import jax
import jax.numpy as jnp
from jax import lax
from jax.experimental import pallas as pl
from jax.experimental.pallas import tpu as pltpu
from jax.experimental.pallas import tpu_sc as plsc

N = 10000
E = 320000
D = 128
R = 8
L = 3
B = 16

NC = 2
NS = 16
LANES = 16

EDGE_CHUNK = 80
E_PER_TILE = E // (NC * NS)
N_CHUNKS_AGG = E_PER_TILE // EDGE_CHUNK
E_PER_CNT_TILE = E // NS
N_CHUNKS_CNT = E_PER_CNT_TILE // EDGE_CHUNK
ROW_CHUNK = 80
N_ROW_CHUNKS = N // ROW_CHUNK
GATHER_CHUNKS = N // EDGE_CHUNK

_MESH = plsc.VectorSubcoreMesh(
    core_axis_name="c", subcore_axis_name="s", num_cores=NC, num_subcores=NS)


@functools.partial(
    pl.kernel,
    out_type=(
        jax.ShapeDtypeStruct((N, D), jnp.float32),
        jax.ShapeDtypeStruct((N * R, D), jnp.float32),
    ),
    mesh=_MESH,
    scratch_types=(
        pltpu.VMEM((EDGE_CHUNK,), jnp.int32),
        pltpu.VMEM((EDGE_CHUNK, D), jnp.float32),
        pltpu.VMEM((EDGE_CHUNK,), jnp.int32),
        pltpu.VMEM((EDGE_CHUNK,), jnp.int32),
        pltpu.VMEM((EDGE_CHUNK, D), jnp.float32),
        pltpu.VMEM((EDGE_CHUNK, D), jnp.float32),
        pltpu.VMEM((EDGE_CHUNK, D), jnp.float32),
        pltpu.VMEM_SHARED((N, D), jnp.float32),
        pltpu.SemaphoreType.DMA,
    ),
)
def _sc_embed_and_count(node_type_hbm, edge_type_hbm, dst_hbm, oh8_hbm,
                        emb_hbm, x0_hbm, invc_hbm,
                        idx_v, rows_v, ti_v, di_v, oh_v, c_v, t2_v,
                        acc_sh, sem):
  c = lax.axis_index("c")
  s = lax.axis_index("s")

  @pl.when(c == 1)
  def _embed():
    def chunk_body(k, _):
      cid = s + k * NS

      @pl.when(cid < GATHER_CHUNKS)
      def _():
        base = pl.multiple_of(cid * EDGE_CHUNK, 8)
        pltpu.sync_copy(node_type_hbm.at[pl.ds(base, EDGE_CHUNK)], idx_v)
        pltpu.async_copy(emb_hbm.at[idx_v], rows_v, sem).wait()
        pltpu.sync_copy(rows_v, x0_hbm.at[pl.ds(base, EDGE_CHUNK)])
      return 0

    lax.fori_loop(0, (GATHER_CHUNKS + NS - 1) // NS, chunk_body, 0)

  @pl.when(c == 0)
  def _zero():
    for j in range(ROW_CHUNK):
      for cc in range(D // LANES):
        t2_v[j, pl.ds(cc * LANES, LANES)] = jnp.zeros((LANES,), jnp.float32)

    def zero_body(k, _):
      cid = s + k * NS

      @pl.when(cid < N_ROW_CHUNKS)
      def _():
        pltpu.sync_copy(t2_v, acc_sh.at[pl.ds(cid * ROW_CHUNK, ROW_CHUNK)])
      return 0

    lax.fori_loop(0, (N_ROW_CHUNKS + NS - 1) // NS, zero_body, 0)

  plsc.subcore_barrier()

  @pl.when(c == 0)
  def _count():
    def chunk_body(k, _):
      base = pl.multiple_of(s * E_PER_CNT_TILE + k * EDGE_CHUNK, 8)
      pltpu.sync_copy(edge_type_hbm.at[pl.ds(base, EDGE_CHUNK)], ti_v)
      pltpu.sync_copy(dst_hbm.at[pl.ds(base, EDGE_CHUNK)], di_v)
      pltpu.async_copy(oh8_hbm.at[ti_v], oh_v, sem).wait()
      pltpu.sync_copy(oh_v, acc_sh.at[di_v], add=True)
      return 0

    lax.fori_loop(0, N_CHUNKS_CNT, chunk_body, 0)

  plsc.subcore_barrier()

  @pl.when(c == 0)
  def _recip():
    def wb_body(k, _):
      cid = s + k * NS

      @pl.when(cid < N_ROW_CHUNKS)
      def _():
        pltpu.sync_copy(acc_sh.at[pl.ds(cid * ROW_CHUNK, ROW_CHUNK)], c_v)

        def batch_body(b, _):
          for j2 in range(ROW_CHUNK):
            src_row = b * (ROW_CHUNK // R) + j2 // R
            blk = pl.ds((j2 % R) * LANES, LANES)
            rec = 1.0 / jnp.maximum(c_v[src_row, blk], 1.0)
            for cc in range(D // LANES):
              t2_v[j2, pl.ds(cc * LANES, LANES)] = rec
          obase = pl.multiple_of(cid * ROW_CHUNK * R + b * ROW_CHUNK, 8)
          pltpu.sync_copy(t2_v, invc_hbm.at[pl.ds(obase, ROW_CHUNK)])
          return 0

        lax.fori_loop(0, R, batch_body, 0)
      return 0

    lax.fori_loop(0, (N_ROW_CHUNKS + NS - 1) // NS, wb_body, 0)


@functools.partial(
    pl.kernel,
    out_type=jax.ShapeDtypeStruct((NC, N, D), jnp.float32),
    mesh=_MESH,
    scratch_types=(
        pltpu.VMEM((EDGE_CHUNK,), jnp.int32),
        pltpu.VMEM((EDGE_CHUNK,), jnp.int32),
        pltpu.VMEM((EDGE_CHUNK,), jnp.int32),
        pltpu.VMEM((EDGE_CHUNK,), jnp.int32),
        pltpu.VMEM((EDGE_CHUNK,), jnp.int32),
        pltpu.VMEM((EDGE_CHUNK,), jnp.int32),
        pltpu.VMEM((EDGE_CHUNK, D), jnp.float32),
        pltpu.VMEM((EDGE_CHUNK, D), jnp.float32),
        pltpu.VMEM((EDGE_CHUNK, D), jnp.float32),
        pltpu.VMEM((EDGE_CHUNK, D), jnp.float32),
        pltpu.VMEM_SHARED((N, D), jnp.float32),
        pltpu.SemaphoreType.DMA,
        pltpu.SemaphoreType.DMA,
        pltpu.SemaphoreType.DMA,
        pltpu.SemaphoreType.DMA,
    ),
)
def _sc_aggregate(hflat_hbm, gidx_hbm, dst_hbm, widx_hbm, invc_hbm, zrows_hbm,
                  out_hbm,
                  gi_v0, gi_v1, di_v0, di_v1, wi_v0, wi_v1, wr_v0, wr_v1,
                  rows_v0, rows_v1, acc_sh, semi0, semi1, semg0, semg1):
  c = lax.axis_index("c")
  s = lax.axis_index("s")
  wid = s * NC + c

  pltpu.sync_copy(zrows_hbm, rows_v0)

  def zero_body(k, _):
    cid = s + k * NS

    @pl.when(cid < N_ROW_CHUNKS)
    def _():
      pltpu.sync_copy(rows_v0, acc_sh.at[pl.ds(cid * ROW_CHUNK, ROW_CHUNK)])
    return 0

  lax.fori_loop(0, (N_ROW_CHUNKS + NS - 1) // NS, zero_body, 0)
  plsc.subcore_barrier()

  bufs = ((gi_v0, di_v0, wi_v0, wr_v0, rows_v0, semi0, semg0),
          (gi_v1, di_v1, wi_v1, wr_v1, rows_v1, semi1, semg1))

  def fire_idx(k, b):
    gi, di, wi, _, _, semi, _ = bufs[b]
    base = pl.multiple_of(wid * E_PER_TILE + k * EDGE_CHUNK, 8)
    return (pltpu.async_copy(gidx_hbm.at[pl.ds(base, EDGE_CHUNK)], gi, semi),
            pltpu.async_copy(dst_hbm.at[pl.ds(base, EDGE_CHUNK)], di, semi),
            pltpu.async_copy(widx_hbm.at[pl.ds(base, EDGE_CHUNK)], wi, semi))

  def fire_gather(b):
    gi, _, wi, wr, rows, _, semg = bufs[b]
    return (pltpu.async_copy(hflat_hbm.at[gi], rows, semg),
            pltpu.async_copy(invc_hbm.at[wi], wr, semg))

  def scale(b):
    _, _, _, wr, rows, _, _ = bufs[b]

    def scale_body(jj, _):
      for j2 in range(8):
        j = jj * 8 + j2
        for cc in range(D // LANES):
          sl = pl.ds(cc * LANES, LANES)
          rows[j, sl] = rows[j, sl] * wr[j, sl]
      return 0

    lax.fori_loop(0, EDGE_CHUNK // 8, scale_body, 0)

  def fire_scatter(b):
    _, di, _, _, rows, semi, _ = bufs[b]
    return pltpu.async_copy(rows, acc_sh.at[di], semi, add=True)

  def pair_body(m, _):
    k0 = m * 2
    i0 = fire_idx(k0, 0)
    i1 = fire_idx(k0 + 1, 1)
    for dsc in i0:
      dsc.wait()
    g0 = fire_gather(0)
    for dsc in i1:
      dsc.wait()
    g1 = fire_gather(1)
    for dsc in g0:
      dsc.wait()
    scale(0)
    sc0 = fire_scatter(0)
    for dsc in g1:
      dsc.wait()
    scale(1)
    sc1 = fire_scatter(1)
    sc0.wait()
    sc1.wait()
    return 0

  lax.fori_loop(0, N_CHUNKS_AGG // 2, pair_body, 0)
  for dsc in fire_idx(N_CHUNKS_AGG - 1, 0):
    dsc.wait()
  for dsc in fire_gather(0):
    dsc.wait()
  scale(0)
  fire_scatter(0).wait()

  plsc.subcore_barrier()

  def wb_body(k, _):
    cid = s + k * NS

    @pl.when(cid < N_ROW_CHUNKS)
    def _():
      sl = pl.ds(cid * ROW_CHUNK, ROW_CHUNK)
      pltpu.sync_copy(acc_sh.at[sl], rows_v0)

      @pl.when(c == 0)
      def _():
        pltpu.sync_copy(rows_v0, out_hbm.at[0, sl])

      @pl.when(c == 1)
      def _():
        pltpu.sync_copy(rows_v0, out_hbm.at[1, sl])
    return 0

  lax.fori_loop(0, (N_ROW_CHUNKS + NS - 1) // NS, wb_body, 0)


ROW_BLK = 1000


def _mm_body(x_ref, w_ref, b_ref, out_ref):
  i = pl.program_id(0)
  h = jnp.dot(x_ref[...], w_ref[0], preferred_element_type=jnp.float32)
  out_ref[0] = h + jnp.where(i == 0, b_ref[...], 0.0)


def _tc_project(x, w_all, bias_row):
  return pl.pallas_call(
      _mm_body,
      grid=(R + 1, N // ROW_BLK),
      in_specs=[
          pl.BlockSpec((ROW_BLK, D), lambda i, j: (j, 0)),
          pl.BlockSpec((1, D, D), lambda i, j: (i, 0, 0)),
          pl.BlockSpec((1, D), lambda i, j: (0, 0)),
      ],
      out_specs=pl.BlockSpec((1, ROW_BLK, D), lambda i, j: (i, j, 0)),
      out_shape=jax.ShapeDtypeStruct((R + 1, N, D), jnp.float32),
  )(x, w_all, bias_row)


def _combine_body(h_ref, a_ref, out_ref):
  out_ref[...] = jnp.maximum(h_ref[0] + a_ref[0] + a_ref[1], 0.0)


def _tc_combine(h9, partials):
  return pl.pallas_call(
      _combine_body,
      grid=(N // ROW_BLK,),
      in_specs=[
          pl.BlockSpec((1, ROW_BLK, D), lambda j: (0, j, 0)),
          pl.BlockSpec((NC, ROW_BLK, D), lambda j: (0, j, 0)),
      ],
      out_specs=pl.BlockSpec((ROW_BLK, D), lambda j: (j, 0)),
      out_shape=jax.ShapeDtypeStruct((N, D), jnp.float32),
  )(h9, partials)


def _pool_heads_body(batch_ref, x_ref, rw1_ref, rb1_ref, rw2_ref, rb2_ref,
                     sw1_ref, sb1_ref, sw2_ref, sb2_ref,
                     risk_ref, safe_ref, g_acc, c_acc):
  j = pl.program_id(0)

  @pl.when(j == 0)
  def _():
    g_acc[...] = jnp.zeros_like(g_acc)
    c_acc[...] = jnp.zeros_like(c_acc)

  bids = batch_ref[0, 0, :]
  sel = (lax.broadcasted_iota(jnp.int32, (B, ROW_BLK), 0)
         == bids[None, :]).astype(jnp.float32)
  g_acc[...] += jnp.dot(sel, x_ref[...], preferred_element_type=jnp.float32)
  c_acc[...] += jnp.broadcast_to(
      jnp.sum(sel, axis=1, keepdims=True), (B, D))

  @pl.when(j == pl.num_programs(0) - 1)
  def _():
    g = g_acc[...] / jnp.maximum(c_acc[...], 1.0)
    r1 = jnp.maximum(
        jnp.dot(g, rw1_ref[...], preferred_element_type=jnp.float32)
        + rb1_ref[...], 0.0)
    risk_ref[...] = jnp.dot(
        r1, rw2_ref[...], preferred_element_type=jnp.float32) + rb2_ref[...]
    s1 = jnp.maximum(
        jnp.dot(g, sw1_ref[...], preferred_element_type=jnp.float32)
        + sb1_ref[...], 0.0)
    safe_ref[...] = jnp.dot(
        s1, sw2_ref[...], preferred_element_type=jnp.float32) + sb2_ref[...]


def _tc_pool_heads(batch3d, x, rw1, rb1, rw2p, rb2f, sw1, sb1, sw2p, sb2f):
  full = lambda shape: pl.BlockSpec(shape, lambda j: tuple(0 for _ in shape))
  return pl.pallas_call(
      _pool_heads_body,
      grid=(N // ROW_BLK,),
      in_specs=[
          pl.BlockSpec((1, 1, ROW_BLK), lambda j: (j, 0, 0)),
          pl.BlockSpec((ROW_BLK, D), lambda j: (j, 0)),
          full((D, D)), full((1, D)), full((D, D)), full((1, D)),
          full((D, D)), full((1, D)), full((D, D)), full((1, D)),
      ],
      out_specs=[full((B, D)), full((B, D))],
      out_shape=[
          jax.ShapeDtypeStruct((B, D), jnp.float32),
          jax.ShapeDtypeStruct((B, D), jnp.float32),
      ],
      scratch_shapes=[
          pltpu.VMEM((B, D), jnp.float32),
          pltpu.VMEM((B, D), jnp.float32),
      ],
  )(batch3d, x, rw1, rb1, rw2p, rb2f, sw1, sb1, sw2p, sb2f)


def kernel(node_type, edge_index, edge_type, batch, node_emb, W_rel, W_root,
           conv_bias, risk_W1, risk_b1, risk_W2, risk_b2, safe_W1, safe_b1,
           safe_W2, safe_b2):
  node_type = node_type.astype(jnp.int32)
  edge_type = edge_type.astype(jnp.int32)
  src = edge_index[0].astype(jnp.int32)
  dst = edge_index[1].astype(jnp.int32)
  gidx = (edge_type + 1) * N + src
  widx = dst * R + edge_type

  zrows_d = jnp.zeros((ROW_CHUNK, D), jnp.float32)
  oh8 = (jnp.arange(D, dtype=jnp.int32)[None, :] // LANES
         == jnp.arange(R, dtype=jnp.int32)[:, None]).astype(jnp.float32)

  x0, invc = _sc_embed_and_count(
      node_type, edge_type, dst, oh8, node_emb.astype(jnp.float32))

  x = x0
  for l in range(L):
    w_all = jnp.concatenate([W_root[l][None], W_rel[l]], axis=0)
    h9 = _tc_project(x, w_all, conv_bias[l][None, :])
    partials = _sc_aggregate(h9.reshape((R + 1) * N, D), gidx, dst, widx,
                             invc, zrows_d)
    x = _tc_combine(h9, partials)

  batch3d = batch.astype(jnp.int32).reshape(N // ROW_BLK, 1, ROW_BLK)
  pad = jnp.zeros((D, D - 1), jnp.float32)
  rw2p = jnp.concatenate([risk_W2, pad], axis=1)
  sw2p = jnp.concatenate([safe_W2, pad], axis=1)
  rb2f = jnp.full((1, D), risk_b2[0], jnp.float32)
  sb2f = jnp.full((1, D), safe_b2[0], jnp.float32)
  risk, safe = _tc_pool_heads(batch3d, x, risk_W1, risk_b1[None, :], rw2p,
                              rb2f, safe_W1, safe_b1[None, :], sw2p, sb2f)
  return risk[:, 0], safe[:, 0]


# speedup vs baseline: 12.5628x; 1.0090x over previous
import functools

import jax
import jax.numpy as jnp
from jax import lax
from jax.experimental import pallas as pl
from jax.experimental.pallas import tpu as pltpu
from jax.experimental.pallas import tpu_sc as plsc

N = 10000
E = 320000
D = 128
R = 8
L = 3
B = 16

NC = 2
NS = 16
LANES = 16

EDGE_CHUNK = 80
E_PER_TILE = E // (NC * NS)
N_CHUNKS_AGG = E_PER_TILE // EDGE_CHUNK
E_PER_CNT_TILE = E // NS
N_CHUNKS_CNT = E_PER_CNT_TILE // EDGE_CHUNK
ROW_CHUNK = 80
N_ROW_CHUNKS = N // ROW_CHUNK
GATHER_CHUNKS = N // EDGE_CHUNK

_MESH = plsc.VectorSubcoreMesh(
    core_axis_name="c", subcore_axis_name="s", num_cores=NC, num_subcores=NS)


@functools.partial(
    pl.kernel,
    out_type=(
        jax.ShapeDtypeStruct((N, D), jnp.float32),
        jax.ShapeDtypeStruct((N * R, D), jnp.float32),
    ),
    mesh=_MESH,
    scratch_types=(
        pltpu.VMEM((EDGE_CHUNK,), jnp.int32),
        pltpu.VMEM((EDGE_CHUNK, D), jnp.float32),
        pltpu.VMEM((EDGE_CHUNK,), jnp.int32),
        pltpu.VMEM((EDGE_CHUNK,), jnp.int32),
        pltpu.VMEM((EDGE_CHUNK,), jnp.int32),
        pltpu.VMEM((EDGE_CHUNK,), jnp.int32),
        pltpu.VMEM((EDGE_CHUNK, D), jnp.float32),
        pltpu.VMEM((EDGE_CHUNK, D), jnp.float32),
        pltpu.VMEM((EDGE_CHUNK, D), jnp.float32),
        pltpu.VMEM_SHARED((N, D), jnp.float32),
        pltpu.SemaphoreType.DMA,
        pltpu.SemaphoreType.DMA,
        pltpu.SemaphoreType.DMA,
        pltpu.SemaphoreType.DMA,
        pltpu.SemaphoreType.DMA,
    ),
)
def _sc_embed_and_count(node_type_hbm, edge_type_hbm, dst_hbm, oh8_hbm,
                        emb_hbm, x0_hbm, invc_hbm,
                        idx_v, rows_v, ti_v0, ti_v1, di_v0, di_v1,
                        oh_v0, oh_v1, t2_v,
                        acc_sh, sem, semc0, semc1, semg0, semg1):
  c_v = rows_v
  c = lax.axis_index("c")
  s = lax.axis_index("s")

  @pl.when(c == 1)
  def _embed():
    def chunk_body(k, _):
      cid = s + k * NS

      @pl.when(cid < GATHER_CHUNKS)
      def _():
        base = pl.multiple_of(cid * EDGE_CHUNK, 8)
        pltpu.sync_copy(node_type_hbm.at[pl.ds(base, EDGE_CHUNK)], idx_v)
        pltpu.async_copy(emb_hbm.at[idx_v], rows_v, sem).wait()
        pltpu.sync_copy(rows_v, x0_hbm.at[pl.ds(base, EDGE_CHUNK)])
      return 0

    lax.fori_loop(0, (GATHER_CHUNKS + NS - 1) // NS, chunk_body, 0)

  @pl.when(c == 0)
  def _zero():
    for j in range(ROW_CHUNK):
      for cc in range(D // LANES):
        t2_v[j, pl.ds(cc * LANES, LANES)] = jnp.zeros((LANES,), jnp.float32)

    def zero_body(k, _):
      cid = s + k * NS

      @pl.when(cid < N_ROW_CHUNKS)
      def _():
        pltpu.sync_copy(t2_v, acc_sh.at[pl.ds(cid * ROW_CHUNK, ROW_CHUNK)])
      return 0

    lax.fori_loop(0, (N_ROW_CHUNKS + NS - 1) // NS, zero_body, 0)

  plsc.subcore_barrier()

  @pl.when(c == 0)
  def _count():
    cbufs = ((ti_v0, di_v0, oh_v0, semc0, semg0),
             (ti_v1, di_v1, oh_v1, semc1, semg1))

    def cfire_idx(k, b):
      ti, di, _, semc, _ = cbufs[b]
      base = pl.multiple_of(s * E_PER_CNT_TILE + k * EDGE_CHUNK, 8)
      return (
          pltpu.async_copy(edge_type_hbm.at[pl.ds(base, EDGE_CHUNK)], ti, semc),
          pltpu.async_copy(dst_hbm.at[pl.ds(base, EDGE_CHUNK)], di, semc))

    def cfire_gather(b):
      ti, _, oh, _, semg = cbufs[b]
      return pltpu.async_copy(oh8_hbm.at[ti], oh, semg)

    def cfire_scatter(b):
      _, di, oh, semc, _ = cbufs[b]
      return pltpu.async_copy(oh, acc_sh.at[di], semc, add=True)

    def pair_body(m, _):
      k0 = m * 2
      i0 = cfire_idx(k0, 0)
      i1 = cfire_idx(k0 + 1, 1)
      for dsc in i0:
        dsc.wait()
      g0 = cfire_gather(0)
      for dsc in i1:
        dsc.wait()
      g1 = cfire_gather(1)
      g0.wait()
      sc0 = cfire_scatter(0)
      g1.wait()
      sc1 = cfire_scatter(1)
      sc0.wait()
      sc1.wait()
      return 0

    lax.fori_loop(0, N_CHUNKS_CNT // 2, pair_body, 0)

  plsc.subcore_barrier()

  @pl.when(c == 0)
  def _recip():
    def wb_body(k, _):
      cid = s + k * NS

      @pl.when(cid < N_ROW_CHUNKS)
      def _():
        pltpu.sync_copy(acc_sh.at[pl.ds(cid * ROW_CHUNK, ROW_CHUNK)], c_v)

        def batch_body(b, _):
          for j2 in range(ROW_CHUNK):
            src_row = b * (ROW_CHUNK // R) + j2 // R
            blk = pl.ds((j2 % R) * LANES, LANES)
            rec = 1.0 / jnp.maximum(c_v[src_row, blk], 1.0)
            for cc in range(D // LANES):
              t2_v[j2, pl.ds(cc * LANES, LANES)] = rec
          obase = pl.multiple_of(cid * ROW_CHUNK * R + b * ROW_CHUNK, 8)
          pltpu.sync_copy(t2_v, invc_hbm.at[pl.ds(obase, ROW_CHUNK)])
          return 0

        lax.fori_loop(0, R, batch_body, 0)
      return 0

    lax.fori_loop(0, (N_ROW_CHUNKS + NS - 1) // NS, wb_body, 0)


@functools.partial(
    pl.kernel,
    out_type=jax.ShapeDtypeStruct((NC, N, D), jnp.float32),
    mesh=_MESH,
    scratch_types=(
        pltpu.VMEM((EDGE_CHUNK,), jnp.int32),
        pltpu.VMEM((EDGE_CHUNK,), jnp.int32),
        pltpu.VMEM((EDGE_CHUNK,), jnp.int32),
        pltpu.VMEM((EDGE_CHUNK,), jnp.int32),
        pltpu.VMEM((EDGE_CHUNK,), jnp.int32),
        pltpu.VMEM((EDGE_CHUNK,), jnp.int32),
        pltpu.VMEM((EDGE_CHUNK, D), jnp.float32),
        pltpu.VMEM((EDGE_CHUNK, D), jnp.float32),
        pltpu.VMEM((EDGE_CHUNK, D), jnp.float32),
        pltpu.VMEM((EDGE_CHUNK, D), jnp.float32),
        pltpu.VMEM_SHARED((N, D), jnp.float32),
        pltpu.SemaphoreType.DMA,
        pltpu.SemaphoreType.DMA,
        pltpu.SemaphoreType.DMA,
        pltpu.SemaphoreType.DMA,
    ),
)
def _sc_aggregate(hflat_hbm, gidx_hbm, dst_hbm, widx_hbm, invc_hbm, zrows_hbm,
                  out_hbm,
                  gi_v0, gi_v1, di_v0, di_v1, wi_v0, wi_v1, wr_v0, wr_v1,
                  rows_v0, rows_v1, acc_sh, semi0, semi1, semg0, semg1):
  c = lax.axis_index("c")
  s = lax.axis_index("s")
  wid = s * NC + c

  pltpu.sync_copy(zrows_hbm, rows_v0)

  def zero_body(k, _):
    cid = s + k * NS

    @pl.when(cid < N_ROW_CHUNKS)
    def _():
      pltpu.sync_copy(rows_v0, acc_sh.at[pl.ds(cid * ROW_CHUNK, ROW_CHUNK)])
    return 0

  lax.fori_loop(0, (N_ROW_CHUNKS + NS - 1) // NS, zero_body, 0)
  plsc.subcore_barrier()

  bufs = ((gi_v0, di_v0, wi_v0, wr_v0, rows_v0, semi0, semg0),
          (gi_v1, di_v1, wi_v1, wr_v1, rows_v1, semi1, semg1))

  def fire_idx(k, b):
    gi, di, wi, _, _, semi, _ = bufs[b]
    base = pl.multiple_of(wid * E_PER_TILE + k * EDGE_CHUNK, 8)
    return (pltpu.async_copy(gidx_hbm.at[pl.ds(base, EDGE_CHUNK)], gi, semi),
            pltpu.async_copy(dst_hbm.at[pl.ds(base, EDGE_CHUNK)], di, semi),
            pltpu.async_copy(widx_hbm.at[pl.ds(base, EDGE_CHUNK)], wi, semi))

  def fire_gather(b):
    gi, _, wi, wr, rows, _, semg = bufs[b]
    return (pltpu.async_copy(hflat_hbm.at[gi], rows, semg),
            pltpu.async_copy(invc_hbm.at[wi], wr, semg))

  def scale(b):
    _, _, _, wr, rows, _, _ = bufs[b]

    def scale_body(jj, _):
      for j2 in range(8):
        j = jj * 8 + j2
        for cc in range(D // LANES):
          sl = pl.ds(cc * LANES, LANES)
          rows[j, sl] = rows[j, sl] * wr[j, sl]
      return 0

    lax.fori_loop(0, EDGE_CHUNK // 8, scale_body, 0)

  def fire_scatter(b):
    _, di, _, _, rows, semi, _ = bufs[b]
    return pltpu.async_copy(rows, acc_sh.at[di], semi, add=True)

  def pair_body(m, _):
    k0 = m * 2
    i0 = fire_idx(k0, 0)
    i1 = fire_idx(k0 + 1, 1)
    for dsc in i0:
      dsc.wait()
    g0 = fire_gather(0)
    for dsc in i1:
      dsc.wait()
    g1 = fire_gather(1)
    for dsc in g0:
      dsc.wait()
    scale(0)
    sc0 = fire_scatter(0)
    for dsc in g1:
      dsc.wait()
    scale(1)
    sc1 = fire_scatter(1)
    sc0.wait()
    sc1.wait()
    return 0

  lax.fori_loop(0, N_CHUNKS_AGG // 2, pair_body, 0)
  for dsc in fire_idx(N_CHUNKS_AGG - 1, 0):
    dsc.wait()
  for dsc in fire_gather(0):
    dsc.wait()
  scale(0)
  fire_scatter(0).wait()

  plsc.subcore_barrier()

  def wb_body(k, _):
    cid = s + k * NS

    @pl.when(cid < N_ROW_CHUNKS)
    def _():
      sl = pl.ds(cid * ROW_CHUNK, ROW_CHUNK)
      pltpu.sync_copy(acc_sh.at[sl], rows_v0)

      @pl.when(c == 0)
      def _():
        pltpu.sync_copy(rows_v0, out_hbm.at[0, sl])

      @pl.when(c == 1)
      def _():
        pltpu.sync_copy(rows_v0, out_hbm.at[1, sl])
    return 0

  lax.fori_loop(0, (N_ROW_CHUNKS + NS - 1) // NS, wb_body, 0)


ROW_BLK = 1000


def _mm_body(x_ref, w_ref, b_ref, out_ref):
  i = pl.program_id(0)
  h = jnp.dot(x_ref[...], w_ref[0], preferred_element_type=jnp.float32)
  out_ref[0] = h + jnp.where(i == 0, b_ref[...], 0.0)


def _tc_project(x, w_all, bias_row):
  return pl.pallas_call(
      _mm_body,
      grid=(R + 1, N // ROW_BLK),
      in_specs=[
          pl.BlockSpec((ROW_BLK, D), lambda i, j: (j, 0)),
          pl.BlockSpec((1, D, D), lambda i, j: (i, 0, 0)),
          pl.BlockSpec((1, D), lambda i, j: (0, 0)),
      ],
      out_specs=pl.BlockSpec((1, ROW_BLK, D), lambda i, j: (i, j, 0)),
      out_shape=jax.ShapeDtypeStruct((R + 1, N, D), jnp.float32),
  )(x, w_all, bias_row)


def _combine_body(h_ref, a_ref, out_ref):
  out_ref[...] = jnp.maximum(h_ref[0] + a_ref[0] + a_ref[1], 0.0)


def _tc_combine(h9, partials):
  return pl.pallas_call(
      _combine_body,
      grid=(N // ROW_BLK,),
      in_specs=[
          pl.BlockSpec((1, ROW_BLK, D), lambda j: (0, j, 0)),
          pl.BlockSpec((NC, ROW_BLK, D), lambda j: (0, j, 0)),
      ],
      out_specs=pl.BlockSpec((ROW_BLK, D), lambda j: (j, 0)),
      out_shape=jax.ShapeDtypeStruct((N, D), jnp.float32),
  )(h9, partials)


def _pool_heads_body(batch_ref, x_ref, rw1_ref, rb1_ref, rw2_ref, rb2_ref,
                     sw1_ref, sb1_ref, sw2_ref, sb2_ref,
                     risk_ref, safe_ref, g_acc, c_acc):
  j = pl.program_id(0)

  @pl.when(j == 0)
  def _():
    g_acc[...] = jnp.zeros_like(g_acc)
    c_acc[...] = jnp.zeros_like(c_acc)

  bids = batch_ref[0, 0, :]
  sel = (lax.broadcasted_iota(jnp.int32, (B, ROW_BLK), 0)
         == bids[None, :]).astype(jnp.float32)
  g_acc[...] += jnp.dot(sel, x_ref[...], preferred_element_type=jnp.float32)
  c_acc[...] += jnp.broadcast_to(
      jnp.sum(sel, axis=1, keepdims=True), (B, D))

  @pl.when(j == pl.num_programs(0) - 1)
  def _():
    g = g_acc[...] / jnp.maximum(c_acc[...], 1.0)
    r1 = jnp.maximum(
        jnp.dot(g, rw1_ref[...], preferred_element_type=jnp.float32)
        + rb1_ref[...], 0.0)
    risk_ref[...] = jnp.dot(
        r1, rw2_ref[...], preferred_element_type=jnp.float32) + rb2_ref[...]
    s1 = jnp.maximum(
        jnp.dot(g, sw1_ref[...], preferred_element_type=jnp.float32)
        + sb1_ref[...], 0.0)
    safe_ref[...] = jnp.dot(
        s1, sw2_ref[...], preferred_element_type=jnp.float32) + sb2_ref[...]


def _tc_pool_heads(batch3d, x, rw1, rb1, rw2p, rb2f, sw1, sb1, sw2p, sb2f):
  full = lambda shape: pl.BlockSpec(shape, lambda j: tuple(0 for _ in shape))
  return pl.pallas_call(
      _pool_heads_body,
      grid=(N // ROW_BLK,),
      in_specs=[
          pl.BlockSpec((1, 1, ROW_BLK), lambda j: (j, 0, 0)),
          pl.BlockSpec((ROW_BLK, D), lambda j: (j, 0)),
          full((D, D)), full((1, D)), full((D, D)), full((1, D)),
          full((D, D)), full((1, D)), full((D, D)), full((1, D)),
      ],
      out_specs=[full((B, D)), full((B, D))],
      out_shape=[
          jax.ShapeDtypeStruct((B, D), jnp.float32),
          jax.ShapeDtypeStruct((B, D), jnp.float32),
      ],
      scratch_shapes=[
          pltpu.VMEM((B, D), jnp.float32),
          pltpu.VMEM((B, D), jnp.float32),
      ],
  )(batch3d, x, rw1, rb1, rw2p, rb2f, sw1, sb1, sw2p, sb2f)


def kernel(node_type, edge_index, edge_type, batch, node_emb, W_rel, W_root,
           conv_bias, risk_W1, risk_b1, risk_W2, risk_b2, safe_W1, safe_b1,
           safe_W2, safe_b2):
  node_type = node_type.astype(jnp.int32)
  edge_type = edge_type.astype(jnp.int32)
  src = edge_index[0].astype(jnp.int32)
  dst = edge_index[1].astype(jnp.int32)
  gidx = (edge_type + 1) * N + src
  widx = dst * R + edge_type

  zrows_d = jnp.zeros((ROW_CHUNK, D), jnp.float32)
  oh8 = (jnp.arange(D, dtype=jnp.int32)[None, :] // LANES
         == jnp.arange(R, dtype=jnp.int32)[:, None]).astype(jnp.float32)

  x0, invc = _sc_embed_and_count(
      node_type, edge_type, dst, oh8, node_emb.astype(jnp.float32))

  x = x0
  for l in range(L):
    w_all = jnp.concatenate([W_root[l][None], W_rel[l]], axis=0)
    h9 = _tc_project(x, w_all, conv_bias[l][None, :])
    partials = _sc_aggregate(h9.reshape((R + 1) * N, D), gidx, dst, widx,
                             invc, zrows_d)
    x = _tc_combine(h9, partials)

  batch3d = batch.astype(jnp.int32).reshape(N // ROW_BLK, 1, ROW_BLK)
  pad = jnp.zeros((D, D - 1), jnp.float32)
  rw2p = jnp.concatenate([risk_W2, pad], axis=1)
  sw2p = jnp.concatenate([safe_W2, pad], axis=1)
  rb2f = jnp.full((1, D), risk_b2[0], jnp.float32)
  sb2f = jnp.full((1, D), safe_b2[0], jnp.float32)
  risk, safe = _tc_pool_heads(batch3d, x, risk_W1, risk_b1[None, :], rw2p,
                              rb2f, safe_W1, safe_b1[None, :], sw2p, sb2f)
  return risk[:, 0], safe[:, 0]


# speedup vs baseline: 21.5981x; 1.7192x over previous
import functools

import jax
import jax.numpy as jnp
from jax import lax
from jax.experimental import pallas as pl
from jax.experimental.pallas import tpu as pltpu
from jax.experimental.pallas import tpu_sc as plsc

N = 10000
E = 320000
D = 128
R = 8
L = 3
B = 16

NC = 2
NS = 16
LANES = 16

EDGE_CHUNK = 80
E_PER_TILE = E // (NC * NS)
N_CHUNKS_AGG = E_PER_TILE // EDGE_CHUNK
E_PER_CNT_TILE = E // NS
N_CHUNKS_CNT = E_PER_CNT_TILE // EDGE_CHUNK
ROW_CHUNK = 80
N_ROW_CHUNKS = N // ROW_CHUNK
GATHER_CHUNKS = N // EDGE_CHUNK

_MESH = plsc.VectorSubcoreMesh(
    core_axis_name="c", subcore_axis_name="s", num_cores=NC, num_subcores=NS)


@functools.partial(
    pl.kernel,
    out_type=(
        jax.ShapeDtypeStruct((N, D), jnp.float32),
        jax.ShapeDtypeStruct((N * R, D), jnp.float32),
    ),
    mesh=_MESH,
    scratch_types=(
        pltpu.VMEM((EDGE_CHUNK,), jnp.int32),
        pltpu.VMEM((EDGE_CHUNK, D), jnp.float32),
        pltpu.VMEM((EDGE_CHUNK,), jnp.int32),
        pltpu.VMEM((EDGE_CHUNK,), jnp.int32),
        pltpu.VMEM((EDGE_CHUNK,), jnp.int32),
        pltpu.VMEM((EDGE_CHUNK,), jnp.int32),
        pltpu.VMEM((EDGE_CHUNK, D), jnp.float32),
        pltpu.VMEM((EDGE_CHUNK, D), jnp.float32),
        pltpu.VMEM((EDGE_CHUNK, D), jnp.float32),
        pltpu.VMEM_SHARED((N, D), jnp.float32),
        pltpu.SemaphoreType.DMA,
        pltpu.SemaphoreType.DMA,
        pltpu.SemaphoreType.DMA,
        pltpu.SemaphoreType.DMA,
        pltpu.SemaphoreType.DMA,
    ),
)
def _sc_embed_and_count(node_type_hbm, edge_type_hbm, dst_hbm, oh8_hbm,
                        emb_hbm, x0_hbm, invc_hbm,
                        idx_v, rows_v, ti_v0, ti_v1, di_v0, di_v1,
                        oh_v0, oh_v1, t2_v,
                        acc_sh, sem, semc0, semc1, semg0, semg1):
  c_v = rows_v
  c = lax.axis_index("c")
  s = lax.axis_index("s")

  @pl.when(c == 1)
  def _embed():
    def chunk_body(k, _):
      cid = s + k * NS

      @pl.when(cid < GATHER_CHUNKS)
      def _():
        base = pl.multiple_of(cid * EDGE_CHUNK, 8)
        pltpu.sync_copy(node_type_hbm.at[pl.ds(base, EDGE_CHUNK)], idx_v)
        pltpu.async_copy(emb_hbm.at[idx_v], rows_v, sem).wait()
        pltpu.sync_copy(rows_v, x0_hbm.at[pl.ds(base, EDGE_CHUNK)])
      return 0

    lax.fori_loop(0, (GATHER_CHUNKS + NS - 1) // NS, chunk_body, 0)

  @pl.when(c == 0)
  def _zero():
    for j in range(ROW_CHUNK):
      for cc in range(D // LANES):
        t2_v[j, pl.ds(cc * LANES, LANES)] = jnp.zeros((LANES,), jnp.float32)

    def zero_body(k, _):
      cid = s + k * NS

      @pl.when(cid < N_ROW_CHUNKS)
      def _():
        pltpu.sync_copy(t2_v, acc_sh.at[pl.ds(cid * ROW_CHUNK, ROW_CHUNK)])
      return 0

    lax.fori_loop(0, (N_ROW_CHUNKS + NS - 1) // NS, zero_body, 0)

  plsc.subcore_barrier()

  @pl.when(c == 0)
  def _count():
    cbufs = ((ti_v0, di_v0, oh_v0, semc0, semg0),
             (ti_v1, di_v1, oh_v1, semc1, semg1))

    def cfire_idx(k, b):
      ti, di, _, semc, _ = cbufs[b]
      base = pl.multiple_of(s * E_PER_CNT_TILE + k * EDGE_CHUNK, 8)
      return (
          pltpu.async_copy(edge_type_hbm.at[pl.ds(base, EDGE_CHUNK)], ti, semc),
          pltpu.async_copy(dst_hbm.at[pl.ds(base, EDGE_CHUNK)], di, semc))

    def cfire_gather(b):
      ti, _, oh, _, semg = cbufs[b]
      return pltpu.async_copy(oh8_hbm.at[ti], oh, semg)

    def cfire_scatter(b):
      _, di, oh, semc, _ = cbufs[b]
      return pltpu.async_copy(oh, acc_sh.at[di], semc, add=True)

    def pair_body(m, _):
      k0 = m * 2
      i0 = cfire_idx(k0, 0)
      i1 = cfire_idx(k0 + 1, 1)
      for dsc in i0:
        dsc.wait()
      g0 = cfire_gather(0)
      for dsc in i1:
        dsc.wait()
      g1 = cfire_gather(1)
      g0.wait()
      sc0 = cfire_scatter(0)
      g1.wait()
      sc1 = cfire_scatter(1)
      sc0.wait()
      sc1.wait()
      return 0

    lax.fori_loop(0, N_CHUNKS_CNT // 2, pair_body, 0)

  plsc.subcore_barrier()

  @pl.when(c == 0)
  def _recip():
    def wb_body(k, _):
      cid = s + k * NS

      @pl.when(cid < N_ROW_CHUNKS)
      def _():
        pltpu.sync_copy(acc_sh.at[pl.ds(cid * ROW_CHUNK, ROW_CHUNK)], c_v)

        def batch_body(b, _):
          for j2 in range(ROW_CHUNK):
            src_row = b * (ROW_CHUNK // R) + j2 // R
            blk = pl.ds((j2 % R) * LANES, LANES)
            rec = 1.0 / jnp.maximum(c_v[src_row, blk], 1.0)
            for cc in range(D // LANES):
              t2_v[j2, pl.ds(cc * LANES, LANES)] = rec
          obase = pl.multiple_of(cid * ROW_CHUNK * R + b * ROW_CHUNK, 8)
          pltpu.sync_copy(t2_v, invc_hbm.at[pl.ds(obase, ROW_CHUNK)])
          return 0

        lax.fori_loop(0, R, batch_body, 0)
      return 0

    lax.fori_loop(0, (N_ROW_CHUNKS + NS - 1) // NS, wb_body, 0)


@functools.partial(
    pl.kernel,
    out_type=jax.ShapeDtypeStruct((NC, N, D), jnp.float32),
    mesh=_MESH,
    scratch_types=(
        pltpu.VMEM((EDGE_CHUNK,), jnp.int32),
        pltpu.VMEM((EDGE_CHUNK,), jnp.int32),
        pltpu.VMEM((EDGE_CHUNK,), jnp.int32),
        pltpu.VMEM((EDGE_CHUNK,), jnp.int32),
        pltpu.VMEM((EDGE_CHUNK,), jnp.int32),
        pltpu.VMEM((EDGE_CHUNK,), jnp.int32),
        pltpu.VMEM((EDGE_CHUNK, D), jnp.float32),
        pltpu.VMEM((EDGE_CHUNK, D), jnp.float32),
        pltpu.VMEM((EDGE_CHUNK, D), jnp.float32),
        pltpu.VMEM((EDGE_CHUNK, D), jnp.float32),
        pltpu.VMEM_SHARED((N, D), jnp.float32),
        pltpu.SemaphoreType.DMA,
        pltpu.SemaphoreType.DMA,
        pltpu.SemaphoreType.DMA,
        pltpu.SemaphoreType.DMA,
    ),
)
def _sc_aggregate(hflat_hbm, gidx_hbm, dst_hbm, widx_hbm, invc_hbm, zrows_hbm,
                  out_hbm,
                  gi_v0, gi_v1, di_v0, di_v1, wi_v0, wi_v1, wr_v0, wr_v1,
                  rows_v0, rows_v1, acc_sh, semi0, semi1, semg0, semg1):
  c = lax.axis_index("c")
  s = lax.axis_index("s")
  wid = s * NC + c

  pltpu.sync_copy(zrows_hbm, rows_v0)

  def zero_body(k, _):
    cid = s + k * NS

    @pl.when(cid < N_ROW_CHUNKS)
    def _():
      pltpu.sync_copy(rows_v0, acc_sh.at[pl.ds(cid * ROW_CHUNK, ROW_CHUNK)])
    return 0

  lax.fori_loop(0, (N_ROW_CHUNKS + NS - 1) // NS, zero_body, 0)
  plsc.subcore_barrier()

  bufs = ((gi_v0, di_v0, wi_v0, wr_v0, rows_v0, semi0, semg0),
          (gi_v1, di_v1, wi_v1, wr_v1, rows_v1, semi1, semg1))

  def fire_idx(k, b):
    gi, di, wi, _, _, semi, _ = bufs[b]
    base = pl.multiple_of(wid * E_PER_TILE + k * EDGE_CHUNK, 8)
    return (pltpu.async_copy(gidx_hbm.at[pl.ds(base, EDGE_CHUNK)], gi, semi),
            pltpu.async_copy(dst_hbm.at[pl.ds(base, EDGE_CHUNK)], di, semi),
            pltpu.async_copy(widx_hbm.at[pl.ds(base, EDGE_CHUNK)], wi, semi))

  def fire_gather(b):
    gi, _, wi, wr, rows, _, semg = bufs[b]
    return (pltpu.async_copy(hflat_hbm.at[gi], rows, semg),
            pltpu.async_copy(invc_hbm.at[wi], wr, semg))

  def scale(b):
    _, _, _, wr, rows, _, _ = bufs[b]

    def scale_body(jj, _):
      for j2 in range(8):
        j = jj * 8 + j2
        for cc in range(D // LANES):
          sl = pl.ds(cc * LANES, LANES)
          rows[j, sl] = rows[j, sl] * wr[j, sl]
      return 0

    lax.fori_loop(0, EDGE_CHUNK // 8, scale_body, 0)

  def fire_scatter(b):
    _, di, _, _, rows, semi, _ = bufs[b]
    return pltpu.async_copy(rows, acc_sh.at[di], semi, add=True)

  def pair_body(m, _):
    k0 = m * 2
    i0 = fire_idx(k0, 0)
    i1 = fire_idx(k0 + 1, 1)
    for dsc in i0:
      dsc.wait()
    g0 = fire_gather(0)
    for dsc in i1:
      dsc.wait()
    g1 = fire_gather(1)
    for dsc in g0:
      dsc.wait()
    scale(0)
    sc0 = fire_scatter(0)
    for dsc in g1:
      dsc.wait()
    scale(1)
    sc1 = fire_scatter(1)
    sc0.wait()
    sc1.wait()
    return 0

  lax.fori_loop(0, N_CHUNKS_AGG // 2, pair_body, 0)
  for dsc in fire_idx(N_CHUNKS_AGG - 1, 0):
    dsc.wait()
  for dsc in fire_gather(0):
    dsc.wait()
  scale(0)
  fire_scatter(0).wait()

  plsc.subcore_barrier()

  def wb_body(k, _):
    cid = s + k * NS

    @pl.when(cid < N_ROW_CHUNKS)
    def _():
      sl = pl.ds(cid * ROW_CHUNK, ROW_CHUNK)
      pltpu.sync_copy(acc_sh.at[sl], rows_v0)

      @pl.when(c == 0)
      def _():
        pltpu.sync_copy(rows_v0, out_hbm.at[0, sl])

      @pl.when(c == 1)
      def _():
        pltpu.sync_copy(rows_v0, out_hbm.at[1, sl])
    return 0

  lax.fori_loop(0, (N_ROW_CHUNKS + NS - 1) // NS, wb_body, 0)


ROW_BLK = 1000


def _mm_body(x_ref, w_ref, b_ref, out_ref):
  i = pl.program_id(0)
  h = jnp.dot(x_ref[...], w_ref[0], preferred_element_type=jnp.float32)
  out_ref[0] = h + jnp.where(i == 0, b_ref[...], 0.0)


def _tc_project(x, w_all, bias_row):
  return pl.pallas_call(
      _mm_body,
      grid=(R + 1, N // ROW_BLK),
      in_specs=[
          pl.BlockSpec((ROW_BLK, D), lambda i, j: (j, 0)),
          pl.BlockSpec((1, D, D), lambda i, j: (i, 0, 0)),
          pl.BlockSpec((1, D), lambda i, j: (0, 0)),
      ],
      out_specs=pl.BlockSpec((1, ROW_BLK, D), lambda i, j: (i, j, 0)),
      out_shape=jax.ShapeDtypeStruct((R + 1, N, D), jnp.float32),
  )(x, w_all, bias_row)


def _combine_body(h_ref, a_ref, out_ref):
  out_ref[...] = jnp.maximum(h_ref[0] + a_ref[0] + a_ref[1], 0.0)


def _tc_combine(h9, partials):
  return pl.pallas_call(
      _combine_body,
      grid=(N // ROW_BLK,),
      in_specs=[
          pl.BlockSpec((1, ROW_BLK, D), lambda j: (0, j, 0)),
          pl.BlockSpec((NC, ROW_BLK, D), lambda j: (0, j, 0)),
      ],
      out_specs=pl.BlockSpec((ROW_BLK, D), lambda j: (j, 0)),
      out_shape=jax.ShapeDtypeStruct((N, D), jnp.float32),
  )(h9, partials)


def _pool_heads_body(batch_ref, x_ref, rw1_ref, rb1_ref, rw2_ref, rb2_ref,
                     sw1_ref, sb1_ref, sw2_ref, sb2_ref,
                     risk_ref, safe_ref, g_acc, c_acc):
  j = pl.program_id(0)

  @pl.when(j == 0)
  def _():
    g_acc[...] = jnp.zeros_like(g_acc)
    c_acc[...] = jnp.zeros_like(c_acc)

  bids = batch_ref[0, 0, :]
  sel = (lax.broadcasted_iota(jnp.int32, (B, ROW_BLK), 0)
         == bids[None, :]).astype(jnp.float32)
  g_acc[...] += jnp.dot(sel, x_ref[...], preferred_element_type=jnp.float32)
  c_acc[...] += jnp.broadcast_to(
      jnp.sum(sel, axis=1, keepdims=True), (B, D))

  @pl.when(j == pl.num_programs(0) - 1)
  def _():
    g = g_acc[...] / jnp.maximum(c_acc[...], 1.0)
    r1 = jnp.maximum(
        jnp.dot(g, rw1_ref[...], preferred_element_type=jnp.float32)
        + rb1_ref[...], 0.0)
    risk_ref[...] = jnp.dot(
        r1, rw2_ref[...], preferred_element_type=jnp.float32) + rb2_ref[...]
    s1 = jnp.maximum(
        jnp.dot(g, sw1_ref[...], preferred_element_type=jnp.float32)
        + sb1_ref[...], 0.0)
    safe_ref[...] = jnp.dot(
        s1, sw2_ref[...], preferred_element_type=jnp.float32) + sb2_ref[...]


def _tc_pool_heads(batch3d, x, rw1, rb1, rw2p, rb2f, sw1, sb1, sw2p, sb2f):
  full = lambda shape: pl.BlockSpec(shape, lambda j: tuple(0 for _ in shape))
  return pl.pallas_call(
      _pool_heads_body,
      grid=(N // ROW_BLK,),
      in_specs=[
          pl.BlockSpec((1, 1, ROW_BLK), lambda j: (j, 0, 0)),
          pl.BlockSpec((ROW_BLK, D), lambda j: (j, 0)),
          full((D, D)), full((1, D)), full((D, D)), full((1, D)),
          full((D, D)), full((1, D)), full((D, D)), full((1, D)),
      ],
      out_specs=[full((B, D)), full((B, D))],
      out_shape=[
          jax.ShapeDtypeStruct((B, D), jnp.float32),
          jax.ShapeDtypeStruct((B, D), jnp.float32),
      ],
      scratch_shapes=[
          pltpu.VMEM((B, D), jnp.float32),
          pltpu.VMEM((B, D), jnp.float32),
      ],
  )(batch3d, x, rw1, rb1, rw2p, rb2f, sw1, sb1, sw2p, sb2f)


def kernel(node_type, edge_index, edge_type, batch, node_emb, W_rel, W_root,
           conv_bias, risk_W1, risk_b1, risk_W2, risk_b2, safe_W1, safe_b1,
           safe_W2, safe_b2):
  node_type = node_type.astype(jnp.int32)
  edge_type = edge_type.astype(jnp.int32)
  src = edge_index[0].astype(jnp.int32)
  dst = edge_index[1].astype(jnp.int32)
  gidx = (edge_type + 1) * N + src
  widx = dst * R + edge_type

  zrows_d = jnp.zeros((ROW_CHUNK, D), jnp.float32)
  OH_REP = 64
  oh8 = (jnp.arange(D, dtype=jnp.int32)[None, :] // LANES
         == jnp.arange(R * OH_REP, dtype=jnp.int32)[:, None] // OH_REP
         ).astype(jnp.float32)
  ti2 = edge_type * OH_REP + (jnp.arange(E, dtype=jnp.int32) & (OH_REP - 1))

  x0, invc = _sc_embed_and_count(
      node_type, ti2, dst, oh8, node_emb.astype(jnp.float32))

  x = x0
  for l in range(L):
    w_all = jnp.concatenate([W_root[l][None], W_rel[l]], axis=0)
    h9 = _tc_project(x, w_all, conv_bias[l][None, :])
    partials = _sc_aggregate(h9.reshape((R + 1) * N, D), gidx, dst, widx,
                             invc, zrows_d)
    x = _tc_combine(h9, partials)

  batch3d = batch.astype(jnp.int32).reshape(N // ROW_BLK, 1, ROW_BLK)
  pad = jnp.zeros((D, D - 1), jnp.float32)
  rw2p = jnp.concatenate([risk_W2, pad], axis=1)
  sw2p = jnp.concatenate([safe_W2, pad], axis=1)
  rb2f = jnp.full((1, D), risk_b2[0], jnp.float32)
  sb2f = jnp.full((1, D), safe_b2[0], jnp.float32)
  risk, safe = _tc_pool_heads(batch3d, x, risk_W1, risk_b1[None, :], rw2p,
                              rb2f, safe_W1, safe_b1[None, :], sw2p, sb2f)
  return risk[:, 0], safe[:, 0]
